# baseline (speedup 1.0000x reference)
import functools

import jax
import jax.numpy as jnp
from jax import lax
from jax.experimental import pallas as pl
from jax.experimental.pallas import tpu as pltpu
from jax.experimental.pallas import tpu_sc as plsc

N = 1_000_000
L = 16
NW = 32
EPW = 31_264
EPW_LAST = N - 31 * EPW
TBL = 119
CH = 15_632
NCH = -(-EPW // CH)


def _make_sc_gather():
  mesh = plsc.VectorSubcoreMesh(core_axis_name="c", subcore_axis_name="s")

  @functools.partial(
      pl.kernel,
      mesh=mesh,
      out_type=(jax.ShapeDtypeStruct((N,), jnp.float32),
                jax.ShapeDtypeStruct((N,), jnp.float32)),
      compiler_params=pltpu.CompilerParams(needs_layout_passes=False),
      scratch_types=[
          pltpu.VMEM((TBL,), jnp.float32),
          pltpu.VMEM((TBL,), jnp.float32),
          pltpu.VMEM((CH,), jnp.int32),
          pltpu.VMEM((CH,), jnp.int32),
          pltpu.VMEM((CH,), jnp.float32),
          pltpu.VMEM((CH,), jnp.float32),
          pltpu.VMEM((CH,), jnp.float32),
          pltpu.VMEM((CH,), jnp.float32),
          pltpu.SemaphoreType.DMA,
          pltpu.SemaphoreType.DMA,
          pltpu.SemaphoreType.DMA,
          pltpu.SemaphoreType.DMA,
          pltpu.SemaphoreType.DMA,
          pltpu.SemaphoreType.DMA,
      ],
  )
  def sc_gather(z_hbm, scale_hbm, shift_hbm, sg_hbm, sh_hbm,
                scale_v, shift_v, z0, z1, g0, g1, h0, h1,
                zs0, zs1, gs0, gs1, hs0, hs1):
    zb, gb, hb = (z0, z1), (g0, g1), (h0, h1)
    zs, gs, hs = (zs0, zs1), (gs0, gs1), (hs0, hs1)
    wid = lax.axis_index("s") * 2 + lax.axis_index("c")
    base = wid * EPW
    cnt = jnp.where(wid == NW - 1, EPW_LAST, EPW)
    def size_of(k):
      return pl.multiple_of(jnp.minimum(CH, cnt - k * CH), L)

    def start_zin(k, b):
      sz = size_of(k)
      pltpu.async_copy(z_hbm.at[pl.ds(base + k * CH, sz)],
                       zb[b].at[pl.ds(0, sz)], zs[b])

    start_zin(0, 0)
    pltpu.sync_copy(scale_hbm, scale_v)
    pltpu.sync_copy(shift_hbm, shift_v)
    for k in range(NCH):
      b = k % 2
      if k + 1 < NCH:
        start_zin(k + 1, 1 - b)
      sz = size_of(k)
      pltpu.make_async_copy(z_hbm.at[pl.ds(base + k * CH, sz)],
                            zb[b].at[pl.ds(0, sz)], zs[b]).wait()
      if k >= 2:
        psz = size_of(k - 2)
        pltpu.make_async_copy(gb[b].at[pl.ds(0, psz)],
                              sg_hbm.at[pl.ds(base + (k - 2) * CH, psz)],
                              gs[b]).wait()
        pltpu.make_async_copy(hb[b].at[pl.ds(0, psz)],
                              sh_hbm.at[pl.ds(base + (k - 2) * CH, psz)],
                              hs[b]).wait()

      z_v, sg_v, sh_v = zb[b], gb[b], hb[b]

      @plsc.parallel_loop(0, sz, L, unroll=8)
      def body(off):
        z = z_v[pl.ds(off, L)]
        sg_v[pl.ds(off, L)] = plsc.load_gather(scale_v, [z])
        sh_v[pl.ds(off, L)] = plsc.load_gather(shift_v, [z])

      pltpu.async_copy(gb[b].at[pl.ds(0, sz)],
                       sg_hbm.at[pl.ds(base + k * CH, sz)], gs[b])
      pltpu.async_copy(hb[b].at[pl.ds(0, sz)],
                       sh_hbm.at[pl.ds(base + k * CH, sz)], hs[b])

    for k in (NCH - 2, NCH - 1):
      b = k % 2
      sz = size_of(k)
      pltpu.make_async_copy(gb[b].at[pl.ds(0, sz)],
                            sg_hbm.at[pl.ds(base + k * CH, sz)], gs[b]).wait()
      pltpu.make_async_copy(hb[b].at[pl.ds(0, sz)],
                            sh_hbm.at[pl.ds(base + k * CH, sz)], hs[b]).wait()

  return sc_gather


_sc_gather = _make_sc_gather()


@jax.jit
def kernel(x, Z, scale_param, shift_param):
  scp = scale_param.astype(jnp.float32).reshape(-1)
  shp = shift_param.astype(jnp.float32).reshape(-1)
  sg, sh = _sc_gather(Z.astype(jnp.int32), scp, shp)
  x = x.astype(jnp.float32)
  return sg.reshape(-1, 1) * x + sh.reshape(-1, 1)


# speedup vs baseline: 1.0024x; 1.0024x over previous
import functools

import jax
import jax.numpy as jnp
from jax import lax
from jax.experimental import pallas as pl
from jax.experimental.pallas import tpu as pltpu
from jax.experimental.pallas import tpu_sc as plsc

N = 1_000_000
L = 16
NW = 32
EPW = 31_264
EPW_LAST = N - 31 * EPW
TBL = 119
CH = 10_432
NCH = -(-EPW // CH)


def _make_sc_gather():
  mesh = plsc.VectorSubcoreMesh(core_axis_name="c", subcore_axis_name="s")

  @functools.partial(
      pl.kernel,
      mesh=mesh,
      out_type=(jax.ShapeDtypeStruct((N,), jnp.float32),
                jax.ShapeDtypeStruct((N,), jnp.float32)),
      compiler_params=pltpu.CompilerParams(needs_layout_passes=False),
      scratch_types=[
          pltpu.VMEM((TBL,), jnp.float32),
          pltpu.VMEM((TBL,), jnp.float32),
          pltpu.VMEM((CH,), jnp.int32),
          pltpu.VMEM((CH,), jnp.int32),
          pltpu.VMEM((CH,), jnp.float32),
          pltpu.VMEM((CH,), jnp.float32),
          pltpu.VMEM((CH,), jnp.float32),
          pltpu.VMEM((CH,), jnp.float32),
          pltpu.SemaphoreType.DMA,
          pltpu.SemaphoreType.DMA,
          pltpu.SemaphoreType.DMA,
          pltpu.SemaphoreType.DMA,
          pltpu.SemaphoreType.DMA,
          pltpu.SemaphoreType.DMA,
      ],
  )
  def sc_gather(z_hbm, scale_hbm, shift_hbm, sg_hbm, sh_hbm,
                scale_v, shift_v, z0, z1, g0, g1, h0, h1,
                zs0, zs1, gs0, gs1, hs0, hs1):
    zb, gb, hb = (z0, z1), (g0, g1), (h0, h1)
    zs, gs, hs = (zs0, zs1), (gs0, gs1), (hs0, hs1)
    wid = lax.axis_index("s") * 2 + lax.axis_index("c")
    base = wid * EPW
    cnt = jnp.where(wid == NW - 1, EPW_LAST, EPW)
    def size_of(k):
      return pl.multiple_of(jnp.minimum(CH, cnt - k * CH), L)

    def start_zin(k, b):
      sz = size_of(k)
      pltpu.async_copy(z_hbm.at[pl.ds(base + k * CH, sz)],
                       zb[b].at[pl.ds(0, sz)], zs[b])

    start_zin(0, 0)
    pltpu.sync_copy(scale_hbm, scale_v)
    pltpu.sync_copy(shift_hbm, shift_v)
    for k in range(NCH):
      b = k % 2
      if k + 1 < NCH:
        start_zin(k + 1, 1 - b)
      sz = size_of(k)
      pltpu.make_async_copy(z_hbm.at[pl.ds(base + k * CH, sz)],
                            zb[b].at[pl.ds(0, sz)], zs[b]).wait()
      if k >= 2:
        psz = size_of(k - 2)
        pltpu.make_async_copy(gb[b].at[pl.ds(0, psz)],
                              sg_hbm.at[pl.ds(base + (k - 2) * CH, psz)],
                              gs[b]).wait()
        pltpu.make_async_copy(hb[b].at[pl.ds(0, psz)],
                              sh_hbm.at[pl.ds(base + (k - 2) * CH, psz)],
                              hs[b]).wait()

      z_v, sg_v, sh_v = zb[b], gb[b], hb[b]

      @plsc.parallel_loop(0, sz, L, unroll=8)
      def body(off):
        z = z_v[pl.ds(off, L)]
        sg_v[pl.ds(off, L)] = plsc.load_gather(scale_v, [z])
        sh_v[pl.ds(off, L)] = plsc.load_gather(shift_v, [z])

      pltpu.async_copy(gb[b].at[pl.ds(0, sz)],
                       sg_hbm.at[pl.ds(base + k * CH, sz)], gs[b])
      pltpu.async_copy(hb[b].at[pl.ds(0, sz)],
                       sh_hbm.at[pl.ds(base + k * CH, sz)], hs[b])

    for k in (NCH - 2, NCH - 1):
      b = k % 2
      sz = size_of(k)
      pltpu.make_async_copy(gb[b].at[pl.ds(0, sz)],
                            sg_hbm.at[pl.ds(base + k * CH, sz)], gs[b]).wait()
      pltpu.make_async_copy(hb[b].at[pl.ds(0, sz)],
                            sh_hbm.at[pl.ds(base + k * CH, sz)], hs[b]).wait()

  return sc_gather


_sc_gather = _make_sc_gather()


@jax.jit
def kernel(x, Z, scale_param, shift_param):
  scp = scale_param.astype(jnp.float32).reshape(-1)
  shp = shift_param.astype(jnp.float32).reshape(-1)
  sg, sh = _sc_gather(Z.astype(jnp.int32), scp, shp)
  x = x.astype(jnp.float32)
  return sg.reshape(-1, 1) * x + sh.reshape(-1, 1)


# speedup vs baseline: 1.0031x; 1.0006x over previous
import functools

import jax
import jax.numpy as jnp
from jax import lax
from jax.experimental import pallas as pl
from jax.experimental.pallas import tpu as pltpu
from jax.experimental.pallas import tpu_sc as plsc

N = 1_000_000
L = 16
NW = 32
EPW = 31_264
EPW_LAST = N - 31 * EPW
TBL = 119
CH = 8_192
NCH = -(-EPW // CH)


def _make_sc_gather():
  mesh = plsc.VectorSubcoreMesh(core_axis_name="c", subcore_axis_name="s")

  @functools.partial(
      pl.kernel,
      mesh=mesh,
      out_type=(jax.ShapeDtypeStruct((N,), jnp.float32),
                jax.ShapeDtypeStruct((N,), jnp.float32)),
      compiler_params=pltpu.CompilerParams(needs_layout_passes=False),
      scratch_types=[
          pltpu.VMEM((TBL,), jnp.float32),
          pltpu.VMEM((TBL,), jnp.float32),
          pltpu.VMEM((CH,), jnp.int32),
          pltpu.VMEM((CH,), jnp.int32),
          pltpu.VMEM((CH,), jnp.float32),
          pltpu.VMEM((CH,), jnp.float32),
          pltpu.VMEM((CH,), jnp.float32),
          pltpu.VMEM((CH,), jnp.float32),
          pltpu.SemaphoreType.DMA,
          pltpu.SemaphoreType.DMA,
          pltpu.SemaphoreType.DMA,
          pltpu.SemaphoreType.DMA,
          pltpu.SemaphoreType.DMA,
          pltpu.SemaphoreType.DMA,
      ],
  )
  def sc_gather(z_hbm, scale_hbm, shift_hbm, sg_hbm, sh_hbm,
                scale_v, shift_v, z0, z1, g0, g1, h0, h1,
                zs0, zs1, gs0, gs1, hs0, hs1):
    zb, gb, hb = (z0, z1), (g0, g1), (h0, h1)
    zs, gs, hs = (zs0, zs1), (gs0, gs1), (hs0, hs1)
    wid = lax.axis_index("s") * 2 + lax.axis_index("c")
    base = wid * EPW
    cnt = jnp.where(wid == NW - 1, EPW_LAST, EPW)
    def size_of(k):
      return pl.multiple_of(jnp.minimum(CH, cnt - k * CH), L)

    def start_zin(k, b):
      sz = size_of(k)
      pltpu.async_copy(z_hbm.at[pl.ds(base + k * CH, sz)],
                       zb[b].at[pl.ds(0, sz)], zs[b])

    start_zin(0, 0)
    pltpu.sync_copy(scale_hbm, scale_v)
    pltpu.sync_copy(shift_hbm, shift_v)
    for k in range(NCH):
      b = k % 2
      if k + 1 < NCH:
        start_zin(k + 1, 1 - b)
      sz = size_of(k)
      pltpu.make_async_copy(z_hbm.at[pl.ds(base + k * CH, sz)],
                            zb[b].at[pl.ds(0, sz)], zs[b]).wait()
      if k >= 2:
        psz = size_of(k - 2)
        pltpu.make_async_copy(gb[b].at[pl.ds(0, psz)],
                              sg_hbm.at[pl.ds(base + (k - 2) * CH, psz)],
                              gs[b]).wait()
        pltpu.make_async_copy(hb[b].at[pl.ds(0, psz)],
                              sh_hbm.at[pl.ds(base + (k - 2) * CH, psz)],
                              hs[b]).wait()

      z_v, sg_v, sh_v = zb[b], gb[b], hb[b]

      @plsc.parallel_loop(0, sz, L, unroll=8)
      def body(off):
        z = z_v[pl.ds(off, L)]
        sg_v[pl.ds(off, L)] = plsc.load_gather(scale_v, [z])
        sh_v[pl.ds(off, L)] = plsc.load_gather(shift_v, [z])

      pltpu.async_copy(gb[b].at[pl.ds(0, sz)],
                       sg_hbm.at[pl.ds(base + k * CH, sz)], gs[b])
      pltpu.async_copy(hb[b].at[pl.ds(0, sz)],
                       sh_hbm.at[pl.ds(base + k * CH, sz)], hs[b])

    for k in (NCH - 2, NCH - 1):
      b = k % 2
      sz = size_of(k)
      pltpu.make_async_copy(gb[b].at[pl.ds(0, sz)],
                            sg_hbm.at[pl.ds(base + k * CH, sz)], gs[b]).wait()
      pltpu.make_async_copy(hb[b].at[pl.ds(0, sz)],
                            sh_hbm.at[pl.ds(base + k * CH, sz)], hs[b]).wait()

  return sc_gather


_sc_gather = _make_sc_gather()


@jax.jit
def kernel(x, Z, scale_param, shift_param):
  scp = scale_param.astype(jnp.float32).reshape(-1)
  shp = shift_param.astype(jnp.float32).reshape(-1)
  sg, sh = _sc_gather(Z.astype(jnp.int32), scp, shp)
  x = x.astype(jnp.float32)
  return sg.reshape(-1, 1) * x + sh.reshape(-1, 1)


# speedup vs baseline: 1.0083x; 1.0052x over previous
import functools

import jax
import jax.numpy as jnp
from jax import lax
from jax.experimental import pallas as pl
from jax.experimental.pallas import tpu as pltpu
from jax.experimental.pallas import tpu_sc as plsc

N = 1_000_000
L = 16
NW = 32
EPW = 31_264
EPW_LAST = N - 31 * EPW
TBL = 119
CH = 8_192
NCH = -(-EPW // CH)


def _make_sc_gather():
  mesh = plsc.VectorSubcoreMesh(core_axis_name="c", subcore_axis_name="s")

  @functools.partial(
      pl.kernel,
      mesh=mesh,
      out_type=(jax.ShapeDtypeStruct((N,), jnp.float32),
                jax.ShapeDtypeStruct((N,), jnp.float32)),
      compiler_params=pltpu.CompilerParams(needs_layout_passes=False, use_tc_tiling_on_sc=False),
      scratch_types=[
          pltpu.VMEM((TBL,), jnp.float32),
          pltpu.VMEM((TBL,), jnp.float32),
          pltpu.VMEM((CH,), jnp.int32),
          pltpu.VMEM((CH,), jnp.int32),
          pltpu.VMEM((CH,), jnp.float32),
          pltpu.VMEM((CH,), jnp.float32),
          pltpu.VMEM((CH,), jnp.float32),
          pltpu.VMEM((CH,), jnp.float32),
          pltpu.SemaphoreType.DMA,
          pltpu.SemaphoreType.DMA,
          pltpu.SemaphoreType.DMA,
          pltpu.SemaphoreType.DMA,
          pltpu.SemaphoreType.DMA,
          pltpu.SemaphoreType.DMA,
      ],
  )
  def sc_gather(z_hbm, scale_hbm, shift_hbm, sg_hbm, sh_hbm,
                scale_v, shift_v, z0, z1, g0, g1, h0, h1,
                zs0, zs1, gs0, gs1, hs0, hs1):
    zb, gb, hb = (z0, z1), (g0, g1), (h0, h1)
    zs, gs, hs = (zs0, zs1), (gs0, gs1), (hs0, hs1)
    wid = lax.axis_index("s") * 2 + lax.axis_index("c")
    base = wid * EPW
    cnt = jnp.where(wid == NW - 1, EPW_LAST, EPW)
    def size_of(k):
      return pl.multiple_of(jnp.minimum(CH, cnt - k * CH), L)

    def start_zin(k, b):
      sz = size_of(k)
      pltpu.async_copy(z_hbm.at[pl.ds(base + k * CH, sz)],
                       zb[b].at[pl.ds(0, sz)], zs[b])

    start_zin(0, 0)
    pltpu.sync_copy(scale_hbm, scale_v)
    pltpu.sync_copy(shift_hbm, shift_v)
    for k in range(NCH):
      b = k % 2
      if k + 1 < NCH:
        start_zin(k + 1, 1 - b)
      sz = size_of(k)
      pltpu.make_async_copy(z_hbm.at[pl.ds(base + k * CH, sz)],
                            zb[b].at[pl.ds(0, sz)], zs[b]).wait()
      if k >= 2:
        psz = size_of(k - 2)
        pltpu.make_async_copy(gb[b].at[pl.ds(0, psz)],
                              sg_hbm.at[pl.ds(base + (k - 2) * CH, psz)],
                              gs[b]).wait()
        pltpu.make_async_copy(hb[b].at[pl.ds(0, psz)],
                              sh_hbm.at[pl.ds(base + (k - 2) * CH, psz)],
                              hs[b]).wait()

      z_v, sg_v, sh_v = zb[b], gb[b], hb[b]

      @plsc.parallel_loop(0, sz, L, unroll=8)
      def body(off):
        z = z_v[pl.ds(off, L)]
        sg_v[pl.ds(off, L)] = plsc.load_gather(scale_v, [z])
        sh_v[pl.ds(off, L)] = plsc.load_gather(shift_v, [z])

      pltpu.async_copy(gb[b].at[pl.ds(0, sz)],
                       sg_hbm.at[pl.ds(base + k * CH, sz)], gs[b])
      pltpu.async_copy(hb[b].at[pl.ds(0, sz)],
                       sh_hbm.at[pl.ds(base + k * CH, sz)], hs[b])

    for k in (NCH - 2, NCH - 1):
      b = k % 2
      sz = size_of(k)
      pltpu.make_async_copy(gb[b].at[pl.ds(0, sz)],
                            sg_hbm.at[pl.ds(base + k * CH, sz)], gs[b]).wait()
      pltpu.make_async_copy(hb[b].at[pl.ds(0, sz)],
                            sh_hbm.at[pl.ds(base + k * CH, sz)], hs[b]).wait()

  return sc_gather


_sc_gather = _make_sc_gather()


@jax.jit
def kernel(x, Z, scale_param, shift_param):
  scp = scale_param.astype(jnp.float32).reshape(-1)
  shp = shift_param.astype(jnp.float32).reshape(-1)
  sg, sh = _sc_gather(Z.astype(jnp.int32), scp, shp)
  x = x.astype(jnp.float32)
  return sg.reshape(-1, 1) * x + sh.reshape(-1, 1)
